# Initial kernel scaffold; baseline (speedup 1.0000x reference)
import jax
import jax.numpy as jnp
from jax.experimental import pallas as pl


def kernel(obs_traj, obs_traj_obs, nei_index, nei_num_index, enc_W, enc_b, dec_W, dec_b, tl_Wih, tl_Whh, tl_bih, tl_bhh, pl_Wih, pl_Whh, pl_bih, pl_bhh, m_W, m_b, v_W, v_b, sp_W, sp_b, p1_W, p1_b, p2_W, p2_b):
    raise NotImplementedError("write your pallas kernel here")


# speedup vs baseline: 1.0262x; 1.0262x over previous
import jax
import jax.numpy as jnp
from jax.experimental import pallas as pl
from jax.experimental.pallas import tpu as pltpu

_N = 512
_OBS = 8
_PRED = 12
_BI = 128
_NEG = -1e30


def _cell_t(x_t, h_t, c_t, wih, whh, bsum):
    g = (jnp.dot(wih, x_t, preferred_element_type=jnp.float32)
         + jnp.dot(whh, h_t, preferred_element_type=jnp.float32) + bsum)
    gi = jax.nn.sigmoid(g[0:8])
    gf = jax.nn.sigmoid(g[8:16])
    gg = jnp.tanh(g[16:24])
    go = jax.nn.sigmoid(g[24:32])
    c2 = gf * c_t + gi * gg
    h2 = go * jnp.tanh(c2)
    return h2, c2


def _body(obs_ref, curr_ref, nei_ref, h0_ref, c0_ref, eps_ref,
          enc_w_ref, enc_b_ref, dec_w_ref, dec_b_ref,
          tl_wih_ref, tl_whh_ref, tl_b_ref,
          pl_wih_ref, pl_whh_ref, pl_b_ref,
          m2_ref, bconst_ref, phi_ref, phj_ref,
          p2_w_ref, p2_b_ref,
          mwh_ref, mwc_ref, vwh_ref, vwc_ref, mb_ref, vb_ref,
          preds_ref, means_ref, lvars_ref,
          ph_s, pc_s, ctx_s, out_s):
    t = pl.program_id(0)

    @pl.when(t == 0)
    def _init():
        h = h0_ref[...]
        c = c0_ref[...]
        for k in range(_OBS):
            x = jax.nn.relu(
                jnp.dot(enc_w_ref[...], obs_ref[k],
                        preferred_element_type=jnp.float32) + enc_b_ref[...])
            h, c = _cell_t(x, h, c, tl_wih_ref[...], tl_whh_ref[...],
                           tl_b_ref[...])
        ph_s[...] = h
        pc_s[...] = jnp.zeros_like(c)
        ctx_s[...] = jnp.zeros_like(ctx_s)
        out_s[...] = jnp.zeros_like(out_s)

    inc = jnp.concatenate([ctx_s[...], out_s[...]], axis=0)
    x = jax.nn.relu(
        jnp.dot(dec_w_ref[...], inc, preferred_element_type=jnp.float32)
        + dec_b_ref[...])
    ph, pc = _cell_t(x, ph_s[...], pc_s[...], pl_wih_ref[...],
                     pl_whh_ref[...], pl_b_ref[...])
    ph_s[...] = ph
    pc_s[...] = pc

    u = jnp.dot(m2_ref[...], curr_ref[...],
                preferred_element_type=jnp.float32)
    a = u + jnp.dot(phi_ref[...], ph, preferred_element_type=jnp.float32)
    b = (-u + jnp.dot(phj_ref[...], ph, preferred_element_type=jnp.float32)
         + bconst_ref[...])
    w2 = p2_w_ref[...]
    b2 = p2_b_ref[...]
    for ib in range(_N // _BI):
        a_blk = a[:, ib * _BI:(ib + 1) * _BI]
        h1 = jax.nn.relu(a_blk[:, :, None] + b[:, None, :])
        h2 = jnp.dot(w2, h1.reshape(64, _BI * _N),
                     preferred_element_type=jnp.float32) + b2
        h2 = h2.reshape(8, _BI, _N)
        mask = nei_ref[0, ib * _BI:(ib + 1) * _BI, :] > 0
        pooled = jnp.max(jnp.where(mask[None], h2, _NEG), axis=2)
        ctx_s[:, ib * _BI:(ib + 1) * _BI] = jax.nn.relu(pooled)

    ctx = ctx_s[...]
    mu = (jnp.dot(mwh_ref[...], ph, preferred_element_type=jnp.float32)
          + jnp.dot(mwc_ref[...], ctx, preferred_element_type=jnp.float32)
          + mb_ref[...])
    lv = (jnp.dot(vwh_ref[...], ph, preferred_element_type=jnp.float32)
          + jnp.dot(vwc_ref[...], ctx, preferred_element_type=jnp.float32)
          + vb_ref[...])
    out = mu + eps_ref[0] * jnp.exp(0.5 * lv)
    out_s[...] = out
    preds_ref[0] = out
    means_ref[0] = mu
    lvars_ref[0] = lv


def kernel(obs_traj, obs_traj_obs, nei_index, nei_num_index, enc_W, enc_b,
           dec_W, dec_b, tl_Wih, tl_Whh, tl_bih, tl_bhh, pl_Wih, pl_Whh,
           pl_bih, pl_bhh, m_W, m_b, v_W, v_b, sp_W, sp_b, p1_W, p1_b,
           p2_W, p2_b):
    f32 = jnp.float32
    curr_t = obs_traj_obs[-1].T
    kinit = jax.random.key(1)
    h0 = jax.random.normal(jax.random.fold_in(kinit, 0), (_N, 8), f32).T
    c0 = jax.random.normal(jax.random.fold_in(kinit, 1), (_N, 8), f32).T
    eps = jnp.stack([
        jax.random.normal(jax.random.fold_in(kinit, 100 + i), (_N, 2), f32).T
        for i in range(_PRED)])

    p_r = p1_W[:, :32]
    p_hj = p1_W[:, 32:40]
    p_hi = p1_W[:, 40:48]
    m2 = p_r @ sp_W
    bconst = (sp_b @ p_r.T + p1_b).reshape(64, 1)
    mwh = jnp.concatenate([m_W[:, :4], jnp.zeros((2, 4), f32)], axis=1)
    mwc = m_W[:, 4:]
    vwh = jnp.concatenate([jnp.zeros((2, 4), f32), v_W[:, :4]], axis=1)
    vwc = v_W[:, 4:]

    obs_t = obs_traj.transpose(0, 2, 1)

    def full(shape):
        nd = len(shape)
        return pl.BlockSpec(shape, lambda t, _n=nd: (0,) * _n)

    in_specs = [
        full((_OBS, 2, _N)),
        full((2, _N)),
        pl.BlockSpec((1, _N, _N), lambda t: (t, 0, 0)),
        full((8, _N)),
        full((8, _N)),
        pl.BlockSpec((1, 2, _N), lambda t: (t, 0, 0)),
        full((16, 2)), full((16, 1)),
        full((16, 10)), full((16, 1)),
        full((32, 16)), full((32, 8)), full((32, 1)),
        full((32, 16)), full((32, 8)), full((32, 1)),
        full((64, 2)), full((64, 1)),
        full((64, 8)), full((64, 8)),
        full((8, 64)), full((8, 1)),
        full((2, 8)), full((2, 8)), full((2, 8)), full((2, 8)),
        full((2, 1)), full((2, 1)),
    ]
    out_specs = [pl.BlockSpec((1, 2, _N), lambda t: (t, 0, 0))] * 3
    out_shape = [jax.ShapeDtypeStruct((_PRED, 2, _N), f32)] * 3

    preds_t, means_t, lvars_t = pl.pallas_call(
        _body,
        grid=(_PRED,),
        in_specs=in_specs,
        out_specs=out_specs,
        out_shape=out_shape,
        scratch_shapes=[
            pltpu.VMEM((8, _N), f32),
            pltpu.VMEM((8, _N), f32),
            pltpu.VMEM((8, _N), f32),
            pltpu.VMEM((2, _N), f32),
        ],
    )(obs_t, curr_t, nei_index, h0, c0, eps,
      enc_W, enc_b.reshape(16, 1), dec_W, dec_b.reshape(16, 1),
      tl_Wih, tl_Whh, (tl_bih + tl_bhh).reshape(32, 1),
      pl_Wih, pl_Whh, (pl_bih + pl_bhh).reshape(32, 1),
      m2, bconst, p_hi, p_hj, p2_W, p2_b.reshape(8, 1),
      mwh, mwc, vwh, vwc, m_b.reshape(2, 1), v_b.reshape(2, 1))

    return (preds_t.transpose(0, 2, 1), means_t.transpose(0, 2, 1),
            lvars_t.transpose(0, 2, 1))


# speedup vs baseline: 1.9783x; 1.9278x over previous
import jax
import jax.numpy as jnp
from jax.experimental import pallas as pl
from jax.experimental.pallas import tpu as pltpu

_N = 512
_OBS = 8
_PRED = 12
_BI = 128
_NEG = -1e30


def _cell_t(x_t, h_t, c_t, wih, whh, bsum):
    g = (jnp.dot(wih, x_t, preferred_element_type=jnp.float32)
         + jnp.dot(whh, h_t, preferred_element_type=jnp.float32) + bsum)
    gi = jax.nn.sigmoid(g[0:8])
    gf = jax.nn.sigmoid(g[8:16])
    gg = jnp.tanh(g[16:24])
    go = jax.nn.sigmoid(g[24:32])
    c2 = gf * c_t + gi * gg
    h2 = go * jnp.tanh(c2)
    return h2, c2


def _body(obs_ref, curr_ref, nei_ref, h0_ref, c0_ref, eps_ref,
          enc_w_ref, enc_b_ref, dec_w_ref, dec_b_ref,
          tl_wih_ref, tl_whh_ref, tl_b_ref,
          pl_wih_ref, pl_whh_ref, pl_b_ref,
          m2_ref, bconst_ref, phi_ref, phj_ref,
          p2_w_ref, p2_b_ref,
          mwh_ref, mwc_ref, vwh_ref, vwc_ref, mb_ref, vb_ref,
          preds_ref, means_ref, lvars_ref,
          ph_s, pc_s, ctx_s, out_s):
    t = pl.program_id(0)

    @pl.when(t == 0)
    def _init():
        h = h0_ref[...]
        c = c0_ref[...]
        for k in range(_OBS):
            x = jax.nn.relu(
                jnp.dot(enc_w_ref[...], obs_ref[k],
                        preferred_element_type=jnp.float32) + enc_b_ref[...])
            h, c = _cell_t(x, h, c, tl_wih_ref[...], tl_whh_ref[...],
                           tl_b_ref[...])
        ph_s[...] = h
        pc_s[...] = jnp.zeros_like(c)
        ctx_s[...] = jnp.zeros_like(ctx_s)
        out_s[...] = jnp.zeros_like(out_s)

    inc = jnp.concatenate([ctx_s[...], out_s[...]], axis=0)
    x = jax.nn.relu(
        jnp.dot(dec_w_ref[...], inc, preferred_element_type=jnp.float32)
        + dec_b_ref[...])
    ph, pc = _cell_t(x, ph_s[...], pc_s[...], pl_wih_ref[...],
                     pl_whh_ref[...], pl_b_ref[...])
    ph_s[...] = ph
    pc_s[...] = pc

    u = jnp.dot(m2_ref[...], curr_ref[...],
                preferred_element_type=jnp.float32)
    a = u + jnp.dot(phi_ref[...], ph, preferred_element_type=jnp.float32)
    b = (-u + jnp.dot(phj_ref[...], ph, preferred_element_type=jnp.float32)
         + bconst_ref[...])
    w2 = p2_w_ref[...]
    b2 = p2_b_ref[...]
    for ib in range(_N // _BI):
        h1 = jnp.concatenate(
            [jax.nn.relu(a[:, i:i + 1] + b)
             for i in range(ib * _BI, (ib + 1) * _BI)], axis=1)
        h2 = jnp.dot(w2, h1,
                     preferred_element_type=jnp.float32) + b2
        h2 = h2.reshape(8, _BI, _N)
        mask = nei_ref[0, ib * _BI:(ib + 1) * _BI, :] > 0
        pooled = jnp.max(jnp.where(mask[None], h2, _NEG), axis=2)
        ctx_s[:, ib * _BI:(ib + 1) * _BI] = jax.nn.relu(pooled)

    ctx = ctx_s[...]
    mu = (jnp.dot(mwh_ref[...], ph, preferred_element_type=jnp.float32)
          + jnp.dot(mwc_ref[...], ctx, preferred_element_type=jnp.float32)
          + mb_ref[...])
    lv = (jnp.dot(vwh_ref[...], ph, preferred_element_type=jnp.float32)
          + jnp.dot(vwc_ref[...], ctx, preferred_element_type=jnp.float32)
          + vb_ref[...])
    out = mu + eps_ref[0] * jnp.exp(0.5 * lv)
    out_s[...] = out
    preds_ref[0] = out
    means_ref[0] = mu
    lvars_ref[0] = lv


def kernel(obs_traj, obs_traj_obs, nei_index, nei_num_index, enc_W, enc_b,
           dec_W, dec_b, tl_Wih, tl_Whh, tl_bih, tl_bhh, pl_Wih, pl_Whh,
           pl_bih, pl_bhh, m_W, m_b, v_W, v_b, sp_W, sp_b, p1_W, p1_b,
           p2_W, p2_b):
    f32 = jnp.float32
    curr_t = obs_traj_obs[-1].T
    kinit = jax.random.key(1)
    h0 = jax.random.normal(jax.random.fold_in(kinit, 0), (_N, 8), f32).T
    c0 = jax.random.normal(jax.random.fold_in(kinit, 1), (_N, 8), f32).T
    eps = jnp.stack([
        jax.random.normal(jax.random.fold_in(kinit, 100 + i), (_N, 2), f32).T
        for i in range(_PRED)])

    p_r = p1_W[:, :32]
    p_hj = p1_W[:, 32:40]
    p_hi = p1_W[:, 40:48]
    m2 = p_r @ sp_W
    bconst = (sp_b @ p_r.T + p1_b).reshape(64, 1)
    mwh = jnp.concatenate([m_W[:, :4], jnp.zeros((2, 4), f32)], axis=1)
    mwc = m_W[:, 4:]
    vwh = jnp.concatenate([jnp.zeros((2, 4), f32), v_W[:, :4]], axis=1)
    vwc = v_W[:, 4:]

    obs_t = obs_traj.transpose(0, 2, 1)

    def full(shape):
        nd = len(shape)
        return pl.BlockSpec(shape, lambda t, _n=nd: (0,) * _n)

    in_specs = [
        full((_OBS, 2, _N)),
        full((2, _N)),
        pl.BlockSpec((1, _N, _N), lambda t: (t, 0, 0)),
        full((8, _N)),
        full((8, _N)),
        pl.BlockSpec((1, 2, _N), lambda t: (t, 0, 0)),
        full((16, 2)), full((16, 1)),
        full((16, 10)), full((16, 1)),
        full((32, 16)), full((32, 8)), full((32, 1)),
        full((32, 16)), full((32, 8)), full((32, 1)),
        full((64, 2)), full((64, 1)),
        full((64, 8)), full((64, 8)),
        full((8, 64)), full((8, 1)),
        full((2, 8)), full((2, 8)), full((2, 8)), full((2, 8)),
        full((2, 1)), full((2, 1)),
    ]
    out_specs = [pl.BlockSpec((1, 2, _N), lambda t: (t, 0, 0))] * 3
    out_shape = [jax.ShapeDtypeStruct((_PRED, 2, _N), f32)] * 3

    preds_t, means_t, lvars_t = pl.pallas_call(
        _body,
        grid=(_PRED,),
        in_specs=in_specs,
        out_specs=out_specs,
        out_shape=out_shape,
        scratch_shapes=[
            pltpu.VMEM((8, _N), f32),
            pltpu.VMEM((8, _N), f32),
            pltpu.VMEM((8, _N), f32),
            pltpu.VMEM((2, _N), f32),
        ],
    )(obs_t, curr_t, nei_index, h0, c0, eps,
      enc_W, enc_b.reshape(16, 1), dec_W, dec_b.reshape(16, 1),
      tl_Wih, tl_Whh, (tl_bih + tl_bhh).reshape(32, 1),
      pl_Wih, pl_Whh, (pl_bih + pl_bhh).reshape(32, 1),
      m2, bconst, p_hi, p_hj, p2_W, p2_b.reshape(8, 1),
      mwh, mwc, vwh, vwc, m_b.reshape(2, 1), v_b.reshape(2, 1))

    return (preds_t.transpose(0, 2, 1), means_t.transpose(0, 2, 1),
            lvars_t.transpose(0, 2, 1))
